# speedup vs baseline: 387.2434x; 387.2434x over previous
import jax
import jax.numpy as jnp
from jax.experimental import pallas as pl

B = 4096
N = 121
NP = 128
H1 = 2048
BB = 512


def _gcn_kernel(x_ref, adjT_ref, w1_ref, w2_ref, out_ref):
    w1 = w1_ref[...]
    w2 = w2_ref[...]
    a_pos = jnp.sum(jnp.maximum(w1, 0.0) * w2)
    a_neg = jnp.sum(jnp.minimum(w1, 0.0) * w2)

    adjT = adjT_ref[...]
    xb = x_ref[...]
    u = jnp.dot(xb, adjT, preferred_element_type=jnp.float32)
    v = a_pos * jnp.maximum(u, 0.0) + a_neg * jnp.minimum(u, 0.0)
    out_ref[...] = jnp.dot(v, adjT, preferred_element_type=jnp.float32)


def kernel(x, adj, W1, W2):
    xp = jnp.pad(x, ((0, 0), (0, NP - N)))
    adjT = jnp.pad(adj, ((0, NP - N), (0, NP - N))).T
    w1 = W1.reshape(16, 128)
    w2 = W2.reshape(16, 128)

    y = pl.pallas_call(
        _gcn_kernel,
        grid=(B // BB,),
        in_specs=[
            pl.BlockSpec((BB, NP), lambda i: (i, 0)),
            pl.BlockSpec((NP, NP), lambda i: (0, 0)),
            pl.BlockSpec((16, 128), lambda i: (0, 0)),
            pl.BlockSpec((16, 128), lambda i: (0, 0)),
        ],
        out_specs=pl.BlockSpec((BB, NP), lambda i: (i, 0)),
        out_shape=jax.ShapeDtypeStruct((B, NP), jnp.float32),
    )(xp, adjT, w1, w2)

    return y[:, :N].reshape(B, 1, N, 1)
